# baseline (speedup 1.0000x reference)
import functools

import jax
import jax.numpy as jnp
from jax import lax
from jax.experimental import pallas as pl
from jax.experimental.pallas import tpu as pltpu
from jax.experimental.pallas import tpu_sc as plsc

_ROWS = 4096
_COLS = 8192
_NPROBE = 128
_NC = 2
_NS = 16
_NW = _NC * _NS
_RPW = _ROWS // _NW
_LANES = 16


def _sc_body(x_hbm, probe_hbm, out_hbm, probe_v, idx_v, buf_v, sem):
    wid = lax.axis_index("s") * _NC + lax.axis_index("c")
    base_row = wid * _RPW

    pltpu.sync_copy(probe_hbm, probe_v)

    def build(k, carry):
        off = (base_row + k) * _COLS
        for m in range(_NPROBE // _LANES):
            sl = pl.ds(m * _LANES, _LANES)
            idx_v[k, sl] = probe_v[sl] + off
        return carry

    lax.fori_loop(0, _RPW, build, 0, unroll=False)

    def fire(k, carry):
        pltpu.async_copy(x_hbm.at[idx_v.at[k]], buf_v.at[k], sem)
        return carry

    lax.fori_loop(0, _RPW, fire, 0, unroll=False)

    def drain(k, carry):
        pltpu.make_async_copy(x_hbm.at[idx_v.at[k]], buf_v.at[k], sem).wait()
        return carry

    lax.fori_loop(0, _RPW, drain, 0, unroll=False)

    pltpu.sync_copy(buf_v, out_hbm.at[pl.ds(base_row, _RPW)])


_sc_gather = functools.partial(
    pl.kernel,
    out_type=jax.ShapeDtypeStruct((_ROWS, _NPROBE), jnp.float32),
    mesh=plsc.VectorSubcoreMesh(
        core_axis_name="c", subcore_axis_name="s",
        num_cores=_NC, num_subcores=_NS,
    ),
    scratch_types=[
        pltpu.VMEM((_NPROBE,), jnp.int32),
        pltpu.VMEM((_RPW, _NPROBE), jnp.int32),
        pltpu.VMEM((_RPW, _NPROBE), jnp.float32),
        pltpu.SemaphoreType.DMA,
    ],
)(_sc_body)


@jax.jit
def kernel(x, probe_idx):
    xflat = x.reshape(_ROWS * _COLS)
    return _sc_gather(xflat, probe_idx.astype(jnp.int32))


# speedup vs baseline: 3.1380x; 3.1380x over previous
import functools

import jax
import jax.numpy as jnp
from jax import lax
from jax.experimental import pallas as pl
from jax.experimental.pallas import tpu as pltpu
from jax.experimental.pallas import tpu_sc as plsc

_ROWS = 4096
_COLS = 8192
_NPROBE = 128
_NC = 2
_NS = 16
_NW = _NC * _NS
_RPW = _ROWS // _NW
_LANES = 16


def _sc_body(x_hbm, probe_hbm, out_hbm, probe_v, idx_v, buf_v, sem):
    wid = lax.axis_index("s") * _NC + lax.axis_index("c")
    base_row = wid * _RPW

    pltpu.sync_copy(probe_hbm, probe_v)

    for m in range(_NPROBE // _LANES):
        sl = pl.ds(m * _LANES, _LANES)
        c = probe_v[sl]
        probe_v[sl] = ((c >> 7) << 10) + (c & 127)

    def build(k, carry):
        i = base_row + k
        off = (i >> 3) * 65536 + (i & 7) * 128
        for m in range(_NPROBE // _LANES):
            sl = pl.ds(m * _LANES, _LANES)
            idx_v[k, sl] = probe_v[sl] + off
        return carry

    lax.fori_loop(0, _RPW, build, 0, unroll=False)

    def fire(k, carry):
        pltpu.async_copy(x_hbm.at[idx_v.at[k]], buf_v.at[k], sem)
        return carry

    lax.fori_loop(0, _RPW, fire, 0, unroll=False)

    def drain(k, carry):
        pltpu.make_async_copy(x_hbm.at[idx_v.at[k]], buf_v.at[k], sem).wait()
        return carry

    lax.fori_loop(0, _RPW, drain, 0, unroll=False)

    pltpu.sync_copy(buf_v, out_hbm.at[pl.ds(base_row, _RPW)])


_sc_gather = functools.partial(
    pl.kernel,
    out_type=jax.ShapeDtypeStruct((_ROWS, _NPROBE), jnp.float32),
    mesh=plsc.VectorSubcoreMesh(
        core_axis_name="c", subcore_axis_name="s",
        num_cores=_NC, num_subcores=_NS,
    ),
    scratch_types=[
        pltpu.VMEM((_NPROBE,), jnp.int32),
        pltpu.VMEM((_RPW, _NPROBE), jnp.int32),
        pltpu.VMEM((_RPW, _NPROBE), jnp.float32),
        pltpu.SemaphoreType.DMA,
    ],
)(_sc_body)


@jax.jit
def kernel(x, probe_idx):
    xflat = (
        x.reshape(_ROWS // 8, 8, _COLS // 128, 128)
        .transpose(0, 2, 1, 3)
        .reshape(_ROWS * _COLS)
    )
    return _sc_gather(xflat, probe_idx.astype(jnp.int32))


# speedup vs baseline: 3.4440x; 1.0975x over previous
import functools

import jax
import jax.numpy as jnp
from jax import lax
from jax.experimental import pallas as pl
from jax.experimental.pallas import tpu as pltpu
from jax.experimental.pallas import tpu_sc as plsc

_ROWS = 4096
_COLS = 8192
_NPROBE = 128
_NC = 2
_NS = 16
_NW = _NC * _NS
_RPW = _ROWS // _NW
_LANES = 16


def _sc_body(x_hbm, probe_hbm, out_hbm, probe_v, idx_v, buf_v, sem):
    wid = lax.axis_index("s") * _NC + lax.axis_index("c")
    base_row = wid * _RPW

    pltpu.sync_copy(probe_hbm, probe_v)

    for m in range(_NPROBE // _LANES):
        sl = pl.ds(m * _LANES, _LANES)
        c = probe_v[sl]
        probe_v[sl] = ((c >> 7) << 10) + (c & 127)

    def build_fire(k, carry):
        i = base_row + k
        off = (i >> 3) * 65536 + (i & 7) * 128
        for m in range(_NPROBE // _LANES):
            sl = pl.ds(m * _LANES, _LANES)
            idx_v[k, sl] = probe_v[sl] + off
        pltpu.async_copy(x_hbm.at[idx_v.at[k]], buf_v.at[k], sem)
        return carry

    lax.fori_loop(0, _RPW, build_fire, 0, unroll=False)

    def drain(k, carry):
        pltpu.make_async_copy(x_hbm.at[idx_v.at[k]], buf_v.at[k], sem).wait()
        return carry

    lax.fori_loop(0, _RPW, drain, 0, unroll=False)

    pltpu.sync_copy(buf_v, out_hbm.at[pl.ds(base_row, _RPW)])


_sc_gather = functools.partial(
    pl.kernel,
    out_type=jax.ShapeDtypeStruct((_ROWS, _NPROBE), jnp.float32),
    mesh=plsc.VectorSubcoreMesh(
        core_axis_name="c", subcore_axis_name="s",
        num_cores=_NC, num_subcores=_NS,
    ),
    scratch_types=[
        pltpu.VMEM((_NPROBE,), jnp.int32),
        pltpu.VMEM((_RPW, _NPROBE), jnp.int32),
        pltpu.VMEM((_RPW, _NPROBE), jnp.float32),
        pltpu.SemaphoreType.DMA,
    ],
)(_sc_body)


@jax.jit
def kernel(x, probe_idx):
    xflat = (
        x.reshape(_ROWS // 8, 8, _COLS // 128, 128)
        .transpose(0, 2, 1, 3)
        .reshape(_ROWS * _COLS)
    )
    return _sc_gather(xflat, probe_idx.astype(jnp.int32))
